# baseline (speedup 1.0000x reference)
import functools

import jax
import jax.numpy as jnp
from jax import lax
from jax.experimental import pallas as pl
from jax.experimental.pallas import tpu as pltpu
from jax.experimental.pallas import tpu_sc as plsc

_B = 1024
_LPAD = 208
_D = 3072
_LANES = 16
_CH = 8
_NBUF = 4
_NC = 2
_NS = 16
_NW = _NC * _NS
_SEQ_PER_W = _B // _NW
_NCH = _LPAD // _CH
_KD = _D // _LANES
_IDSPAN = 131072

_mesh = plsc.VectorSubcoreMesh(core_axis_name="c", subcore_axis_name="s")


@functools.partial(
    pl.kernel,
    mesh=_mesh,
    out_type=jax.ShapeDtypeStruct((_B, _D), jnp.float32),
    scratch_types=[
        pltpu.VMEM((_SEQ_PER_W * _LPAD,), jnp.int32),
        pltpu.VMEM((_SEQ_PER_W * _LPAD + _LANES,), jnp.float32),
        pltpu.VMEM((_NBUF, _CH, _D), jnp.float32),
        pltpu.VMEM((_D,), jnp.float32),
        pltpu.SemaphoreType.DMA,
        pltpu.SemaphoreType.DMA,
        pltpu.SemaphoreType.DMA,
        pltpu.SemaphoreType.DMA,
    ],
)
def _pooled_embed(
    ids_hbm, w_hbm, table_hbm, out_hbm,
    ids_v, w_v, ring_v, acc_v,
    sem0, sem1, sem2, sem3,
):
    sems = (sem0, sem1, sem2, sem3)
    wid = lax.axis_index("s") * _NC + lax.axis_index("c")
    base = wid * _SEQ_PER_W

    def gather(o, j):
        pltpu.async_copy(
            table_hbm.at[ids_v.at[pl.ds(o + j * _CH, _CH)]],
            ring_v.at[j % _NBUF],
            sems[j % _NBUF],
        )

    def gather_wait(j):
        pltpu.make_async_copy(
            table_hbm.at[ids_v.at[pl.ds(0, _CH)]],
            ring_v.at[j % _NBUF],
            sems[j % _NBUF],
        ).wait()

    def accumulate(j, wvec, first):
        rows = ring_v.at[j % _NBUF]
        ws = [wvec[r] for r in range(_CH)]

        def acc_k(k, c):
            c0 = k * _LANES
            if first:
                v = rows[0, pl.ds(c0, _LANES)] * ws[0]
                lo = 1
            else:
                v = acc_v[pl.ds(c0, _LANES)]
                lo = 0
            for r in range(lo, _CH):
                v = v + rows[r, pl.ds(c0, _LANES)] * ws[r]
            acc_v[pl.ds(c0, _LANES)] = v
            return c

        lax.fori_loop(0, _KD, acc_k, 0)

    pltpu.sync_copy(
        ids_hbm.at[pl.ds(base * _LPAD, _SEQ_PER_W * _LPAD)], ids_v
    )
    pltpu.sync_copy(
        w_hbm.at[pl.ds(base * _LPAD, _SEQ_PER_W * _LPAD)],
        w_v.at[pl.ds(0, _SEQ_PER_W * _LPAD)],
    )

    def per_seq(s, carry):
        g = base + s
        o = s * _LPAD

        dsum = jnp.zeros((_LANES,), jnp.float32)
        for j in range(_LPAD // _LANES):
            dsum = dsum + w_v[pl.ds(o + j * _LANES, _LANES)]
        total = dsum[0]
        for r in range(1, _LANES):
            total = total + dsum[r]

        for d in range(_NBUF):
            @pl.when(total > jnp.float32(d * _CH))
            def _(d=d):
                gather(o, d)

        @pl.when(total == 0.0)
        def _():
            def zero_k(k, c):
                acc_v[pl.ds(k * _LANES, _LANES)] = jnp.zeros(
                    (_LANES,), jnp.float32
                )
                return c

            lax.fori_loop(0, _KD, zero_k, 0)

        for j in range(_NCH):
            any_active = total > jnp.float32(j * _CH)
            if j + _NBUF < _NCH:
                @pl.when(total > jnp.float32((j + _NBUF) * _CH))
                def _(j=j):
                    gather(o, j + _NBUF)

            @pl.when(any_active)
            def _(j=j):
                gather_wait(j)
                wvec = w_v[pl.ds(o + j * _CH, _LANES)]
                accumulate(j, wvec, first=(j == 0))

        denom = jnp.maximum(total, 1e-6)
        rv = 1.0 / jnp.full((_LANES,), denom, jnp.float32)

        def scale_k(k, c):
            c0 = k * _LANES
            acc_v[pl.ds(c0, _LANES)] = acc_v[pl.ds(c0, _LANES)] * rv
            return c

        lax.fori_loop(0, _KD, scale_k, 0)
        pltpu.sync_copy(acc_v, out_hbm.at[g])
        return carry

    lax.fori_loop(0, _SEQ_PER_W, per_seq, 0)


def kernel(input_ids, attention_mask, table):
    vocab = table.shape[0]
    ids = jnp.clip(jnp.asarray(input_ids, jnp.int32), 0, vocab - 1)
    active = attention_mask != 0
    key = jnp.where(active, ids, ids + _IDSPAN)
    pad = _LPAD - key.shape[1]
    key = jnp.pad(key, ((0, 0), (0, pad)), constant_values=2 * _IDSPAN)
    key = jnp.sort(key, axis=1)
    ids_sorted = (key % _IDSPAN).reshape(-1)
    w_sorted = (key < _IDSPAN).astype(jnp.float32).reshape(-1)
    return _pooled_embed(ids_sorted, w_sorted, table)


# speedup vs baseline: 1.1303x; 1.1303x over previous
import functools

import jax
import jax.numpy as jnp
from jax import lax
from jax.experimental import pallas as pl
from jax.experimental.pallas import tpu as pltpu
from jax.experimental.pallas import tpu_sc as plsc

_B = 1024
_LPAD = 208
_D = 3072
_LANES = 16
_NC = 2
_NS = 16
_NW = _NC * _NS
_SEQ_PER_W = _B // _NW
_NCH = _LPAD // _LANES
_KD = _D // _LANES
_IDSPAN = 131072

_mesh = plsc.VectorSubcoreMesh(core_axis_name="c", subcore_axis_name="s")


@functools.partial(
    pl.kernel,
    mesh=_mesh,
    out_type=jax.ShapeDtypeStruct((_B, _D), jnp.float32),
    scratch_types=[
        pltpu.VMEM((_SEQ_PER_W * _LPAD,), jnp.int32),
        pltpu.VMEM((_SEQ_PER_W * _LPAD,), jnp.float32),
        pltpu.VMEM((_LANES, _D), jnp.float32),
        pltpu.VMEM((_LANES, _D), jnp.float32),
        pltpu.VMEM((_D,), jnp.float32),
        pltpu.SemaphoreType.DMA,
        pltpu.SemaphoreType.DMA,
    ],
)
def _pooled_embed(
    ids_hbm, w_hbm, table_hbm, out_hbm,
    ids_v, w_v, rows_a, rows_b, acc_v,
    sem_a, sem_b,
):
    wid = lax.axis_index("s") * _NC + lax.axis_index("c")
    base = wid * _SEQ_PER_W

    def gather(o, j, rows, sem):
        pltpu.async_copy(
            table_hbm.at[ids_v.at[pl.ds(o + j * _LANES, _LANES)]], rows, sem
        )

    def gather_wait(rows, sem):
        pltpu.make_async_copy(
            table_hbm.at[ids_v.at[pl.ds(0, _LANES)]], rows, sem
        ).wait()

    def accumulate(rows, wvec):
        ws = [wvec[r] for r in range(_LANES)]

        def acc_k(k, c):
            c0 = k * _LANES
            v = acc_v[pl.ds(c0, _LANES)]
            for r in range(_LANES):
                v = v + rows[r, pl.ds(c0, _LANES)] * ws[r]
            acc_v[pl.ds(c0, _LANES)] = v
            return c

        lax.fori_loop(0, _KD, acc_k, 0)

    pltpu.sync_copy(
        ids_hbm.at[pl.ds(base * _LPAD, _SEQ_PER_W * _LPAD)], ids_v
    )
    pltpu.sync_copy(
        w_hbm.at[pl.ds(base * _LPAD, _SEQ_PER_W * _LPAD)], w_v
    )

    def per_seq(s, carry):
        g = base + s
        o = s * _LPAD

        dsum = jnp.zeros((_LANES,), jnp.float32)
        for j in range(_NCH):
            dsum = dsum + w_v[pl.ds(o + j * _LANES, _LANES)]
        total = dsum[0]
        for r in range(1, _LANES):
            total = total + dsum[r]

        def zero_k(k, c):
            acc_v[pl.ds(k * _LANES, _LANES)] = jnp.zeros((_LANES,), jnp.float32)
            return c

        lax.fori_loop(0, _KD, zero_k, 0)

        @pl.when(total > 0.0)
        def _():
            gather(o, 0, rows_a, sem_a)

        for j in range(_NCH):
            cur, csem = (rows_a, sem_a) if j % 2 == 0 else (rows_b, sem_b)
            any_active = total > jnp.float32(j * _LANES)
            if j + 1 < _NCH:
                nrows, nsem = (rows_a, sem_a) if (j + 1) % 2 == 0 else (rows_b, sem_b)

                @pl.when(total > jnp.float32((j + 1) * _LANES))
                def _(nrows=nrows, nsem=nsem, j=j):
                    gather(o, j + 1, nrows, nsem)

            @pl.when(any_active)
            def _(cur=cur, csem=csem, j=j):
                gather_wait(cur, csem)
                accumulate(cur, w_v[pl.ds(o + j * _LANES, _LANES)])

        denom = jnp.maximum(total, 1e-6)
        rv = 1.0 / jnp.full((_LANES,), denom, jnp.float32)

        def scale_k(k, c):
            c0 = k * _LANES
            acc_v[pl.ds(c0, _LANES)] = acc_v[pl.ds(c0, _LANES)] * rv
            return c

        lax.fori_loop(0, _KD, scale_k, 0)
        pltpu.sync_copy(acc_v, out_hbm.at[g])
        return carry

    lax.fori_loop(0, _SEQ_PER_W, per_seq, 0)


def kernel(input_ids, attention_mask, table):
    vocab = table.shape[0]
    ids = jnp.clip(jnp.asarray(input_ids, jnp.int32), 0, vocab - 1)
    active = attention_mask != 0
    key = jnp.where(active, ids, ids + _IDSPAN)
    pad = _LPAD - key.shape[1]
    key = jnp.pad(key, ((0, 0), (0, pad)), constant_values=2 * _IDSPAN)
    key = jnp.sort(key, axis=1)
    ids_sorted = (key % _IDSPAN).reshape(-1)
    w_sorted = (key < _IDSPAN).astype(jnp.float32).reshape(-1)
    return _pooled_embed(ids_sorted, w_sorted, table)


# speedup vs baseline: 1.1998x; 1.0615x over previous
import functools

import jax
import jax.numpy as jnp
from jax import lax
from jax.experimental import pallas as pl
from jax.experimental.pallas import tpu as pltpu
from jax.experimental.pallas import tpu_sc as plsc

_B = 1024
_LPAD = 224
_NCH = 13
_D = 3072
_LANES = 16
_NC = 2
_NS = 16
_NW = _NC * _NS
_SEQ_PER_W = _B // _NW
_NPAIR = _LPAD // (2 * _LANES)
_KD = _D // _LANES
_IDSPAN = 131072
_STAGE = _SEQ_PER_W * _LPAD

_mesh = plsc.VectorSubcoreMesh(core_axis_name="c", subcore_axis_name="s")


@functools.partial(
    pl.kernel,
    mesh=_mesh,
    out_type=jax.ShapeDtypeStruct((_B, _D), jnp.float32),
    scratch_types=[
        pltpu.VMEM((_STAGE + 2 * _LANES,), jnp.int32),
        pltpu.VMEM((_STAGE + 2 * _LANES,), jnp.float32),
        pltpu.VMEM((_LANES, _D), jnp.float32),
        pltpu.VMEM((_LANES, _D), jnp.float32),
        pltpu.VMEM((_D,), jnp.float32),
        pltpu.SMEM((_SEQ_PER_W + 1,), jnp.float32),
        pltpu.SemaphoreType.DMA,
        pltpu.SemaphoreType.DMA,
    ],
)
def _pooled_embed(
    ids_hbm, w_hbm, table_hbm, out_hbm,
    ids_v, w_v, rows_a, rows_b, acc_v, cnt_sm,
    sem_a, sem_b,
):
    wid = lax.axis_index("s") * _NC + lax.axis_index("c")
    base = wid * _SEQ_PER_W

    def gather(off, rows, sem):
        pltpu.async_copy(
            table_hbm.at[ids_v.at[pl.ds(off, _LANES)]], rows, sem
        )

    def gather_wait(rows, sem):
        pltpu.make_async_copy(
            table_hbm.at[ids_v.at[pl.ds(0, _LANES)]], rows, sem
        ).wait()

    def accumulate(rows, wvec, first):
        ws = [wvec[r] for r in range(_LANES)]

        def acc_k(k, c):
            c0 = k * _LANES
            if first:
                v = rows[0, pl.ds(c0, _LANES)] * ws[0]
                lo = 1
            else:
                v = acc_v[pl.ds(c0, _LANES)]
                lo = 0
            for r in range(lo, _LANES):
                v = v + rows[r, pl.ds(c0, _LANES)] * ws[r]
            acc_v[pl.ds(c0, _LANES)] = v
            return c

        lax.fori_loop(0, _KD, acc_k, 0)

    pltpu.sync_copy(
        ids_hbm.at[pl.ds(base * _LPAD, _STAGE)],
        ids_v.at[pl.ds(0, _STAGE)],
    )
    pltpu.sync_copy(
        w_hbm.at[pl.ds(base * _LPAD, _STAGE)],
        w_v.at[pl.ds(0, _STAGE)],
    )

    def count_seq(s, carry):
        o = s * _LPAD
        dsum = jnp.zeros((_LANES,), jnp.float32)
        for j in range(_NCH):
            dsum = dsum + w_v[pl.ds(o + j * _LANES, _LANES)]
        total = dsum[0]
        for r in range(1, _LANES):
            total = total + dsum[r]
        cnt_sm[s] = total
        return carry

    lax.fori_loop(0, _SEQ_PER_W, count_seq, 0)
    cnt_sm[_SEQ_PER_W] = 0.0

    t0 = cnt_sm[0]

    @pl.when(t0 > 0.0)
    def _():
        gather(0, rows_a, sem_a)

    @pl.when(t0 > jnp.float32(_LANES))
    def _():
        gather(_LANES, rows_b, sem_b)

    def per_seq(s, carry):
        g = base + s
        o = s * _LPAD
        onext = o + _LPAD
        t = cnt_sm[s]
        tn = cnt_sm[s + 1]

        for p in range(_NPAIR):
            for half, (rows, sem) in enumerate(((rows_a, sem_a), (rows_b, sem_b))):
                c = 2 * p + half

                @pl.when(t > jnp.float32(c * _LANES))
                def _(c=c, rows=rows, sem=sem):
                    gather_wait(rows, sem)
                    wvec = w_v[pl.ds(o + c * _LANES, _LANES)]
                    accumulate(rows, wvec, first=(c == 0))

                cn = c + 2
                if cn < _NCH:
                    @pl.when(t > jnp.float32(cn * _LANES))
                    def _(cn=cn, rows=rows, sem=sem):
                        gather(o + cn * _LANES, rows, sem)
                elif cn >= 2 * _NPAIR:
                    nxt = cn - 2 * _NPAIR

                    @pl.when(tn > jnp.float32(nxt * _LANES))
                    def _(nxt=nxt, rows=rows, sem=sem):
                        gather(onext + nxt * _LANES, rows, sem)

        @pl.when(t == 0.0)
        def _():
            def zero_k(k, c):
                acc_v[pl.ds(k * _LANES, _LANES)] = jnp.zeros(
                    (_LANES,), jnp.float32
                )
                return c

            lax.fori_loop(0, _KD, zero_k, 0)

        denom = jnp.maximum(t, 1e-6)
        rv = 1.0 / jnp.full((_LANES,), denom, jnp.float32)

        def scale_k(k, c):
            c0 = k * _LANES
            acc_v[pl.ds(c0, _LANES)] = acc_v[pl.ds(c0, _LANES)] * rv
            return c

        lax.fori_loop(0, _KD, scale_k, 0)
        pltpu.sync_copy(acc_v, out_hbm.at[g])
        return carry

    lax.fori_loop(0, _SEQ_PER_W, per_seq, 0)


def kernel(input_ids, attention_mask, table):
    vocab = table.shape[0]
    ids = jnp.clip(jnp.asarray(input_ids, jnp.int32), 0, vocab - 1)
    active = attention_mask != 0
    key = jnp.where(active, ids, ids + _IDSPAN)
    pad = _LPAD - key.shape[1]
    key = jnp.pad(key, ((0, 0), (0, pad)), constant_values=2 * _IDSPAN)
    key = jnp.sort(key, axis=1)
    ids_sorted = (key % _IDSPAN).reshape(-1)
    w_sorted = (key < _IDSPAN).astype(jnp.float32).reshape(-1)
    return _pooled_embed(ids_sorted, w_sorted, table)
